# baseline (speedup 1.0000x reference)
import jax
import jax.numpy as jnp
import numpy as np
from jax.experimental import pallas as pl
from jax.experimental.pallas import tpu as pltpu

_BINS = 10
_ROWS = 16384
_COLS = 1024
_BLK = 512
_PAR = 2
_STEPS = _ROWS // _BLK // _PAR
_CH = 16
_NCH = _BLK // _CH
_SCALE = float(np.float32(_BINS - 0.0001))


def _fold(v):
    acc = v[:, 0:128]
    for k in range(1, _COLS // 128):
        acc = acc + v[:, k * 128:(k + 1) * 128]
    while acc.shape[0] > 8:
        h = acc.shape[0] // 2
        acc = acc[0:h, :] + acc[h:, :]
    return acc


def _ghm_kernel(x_ref, t_ref, out_ref, accL_ref, accC_ref):
    i = pl.program_id(1)

    def chunk_body(c, carry):
        accL, accC, tot = carry
        x = x_ref[pl.ds(c * _CH, _CH), :]
        t = t_ref[pl.ds(c * _CH, _CH), :]
        enax = jnp.exp(-jnp.abs(x))
        r = 1.0 / (1.0 + enax)
        sg = jnp.where(x >= 0.0, r, 1.0 - r)
        y = jnp.abs(sg - t) * _SCALE
        idx = jnp.floor(y).astype(jnp.int32)
        loss = jnp.maximum(x, 0.0) - x * t + jnp.log1p(enax)
        newL, newC = [], []
        for b in range(1, _BINS):
            m = idx == b
            newL.append(accL[b - 1] + _fold(jnp.where(m, loss, 0.0)))
            newC.append(accC[b - 1] + _fold(jnp.where(m, 1.0, 0.0)))
        return newL, newC, tot + _fold(loss)

    @pl.when(i == 0)
    def _init():
        accL_ref[...] = jnp.zeros_like(accL_ref)
        accC_ref[...] = jnp.zeros_like(accC_ref)

    accL0 = [accL_ref[b] for b in range(_BINS - 1)]
    accC0 = [accC_ref[b] for b in range(_BINS - 1)]
    tot0 = accC_ref[_BINS - 1]
    accL, accC, tot = jax.lax.fori_loop(
        0, _NCH, chunk_body, (accL0, accC0, tot0)
    )
    for b in range(_BINS - 1):
        accL_ref[b] = accL[b]
        accC_ref[b] = accC[b]
    accC_ref[_BINS - 1] = tot

    @pl.when(i == _STEPS - 1)
    def _final():
        for b in range(_BINS - 1):
            out_ref[0, 0, b + 1] = jnp.sum(accC_ref[b])
            out_ref[0, 1, b + 1] = jnp.sum(accL_ref[b])
        out_ref[0, 0, 0] = 0.0
        out_ref[0, 1, 0] = jnp.sum(accC_ref[_BINS - 1])


def kernel(x, target):
    parts = pl.pallas_call(
        _ghm_kernel,
        grid=(_PAR, _STEPS),
        in_specs=[
            pl.BlockSpec((_BLK, _COLS), lambda o, i: (o * _STEPS + i, 0)),
            pl.BlockSpec((_BLK, _COLS), lambda o, i: (o * _STEPS + i, 0)),
        ],
        out_specs=pl.BlockSpec(
            (1, 2, _BINS), lambda o, i: (o, 0, 0), memory_space=pltpu.SMEM
        ),
        out_shape=jax.ShapeDtypeStruct((_PAR, 2, _BINS), jnp.float32),
        scratch_shapes=[
            pltpu.VMEM((_BINS, 8, 128), jnp.float32),
            pltpu.VMEM((_BINS, 8, 128), jnp.float32),
        ],
        compiler_params=pltpu.CompilerParams(
            dimension_semantics=("parallel", "arbitrary"),
        ),
    )(x, target)
    cnt = parts[:, 0, :].sum(axis=0)
    ls = parts[:, 1, :].sum(axis=0)
    m = jnp.float32(_ROWS * _COLS)
    cnt = cnt.at[0].set(m - jnp.sum(cnt))
    ls = ls.at[0].set(ls[0] - jnp.sum(ls[1:]))
    ne = jnp.sum(cnt > 0.0).astype(jnp.float32)
    gd = jnp.maximum(cnt * ne, 1e-6)
    beta = jnp.float32(_ROWS) / gd
    return jnp.sum(beta * ls) / m


# speedup vs baseline: 1.0224x; 1.0224x over previous
import jax
import jax.numpy as jnp
import numpy as np
from jax.experimental import pallas as pl
from jax.experimental.pallas import tpu as pltpu

_BINS = 10
_ROWS = 16384
_COLS = 1024
_BLK = 512
_PAR = 1
_STEPS = _ROWS // _BLK // _PAR
_CH = 16
_UNROLL = 2
_NCH = _BLK // (_CH * _UNROLL)
_SCALE = float(np.float32(_BINS - 0.0001))


def _fold(v):
    acc = v[:, 0:128]
    for k in range(1, _COLS // 128):
        acc = acc + v[:, k * 128:(k + 1) * 128]
    while acc.shape[0] > 8:
        h = acc.shape[0] // 2
        acc = acc[0:h, :] + acc[h:, :]
    return acc


def _ghm_kernel(x_ref, t_ref, out_ref, accL_ref, accC_ref):
    i = pl.program_id(1)

    def chunk_body(c, carry):
        accL, accC, tot = carry
        for sub in range(_UNROLL):
            base = (c * _UNROLL + sub) * _CH
            x = x_ref[pl.ds(base, _CH), :]
            t = t_ref[pl.ds(base, _CH), :]
            enax = jnp.exp(-jnp.abs(x))
            r = 1.0 / (1.0 + enax)
            sg = jnp.where(x >= 0.0, r, 1.0 - r)
            y = jnp.abs(sg - t) * _SCALE
            idx = jnp.floor(y).astype(jnp.int32)
            loss = jnp.maximum(x, 0.0) - x * t + jnp.log1p(enax)
            newL, newC = [], []
            for b in range(1, _BINS):
                m = idx == b
                newL.append(accL[b - 1] + _fold(jnp.where(m, loss, 0.0)))
                newC.append(accC[b - 1] + _fold(jnp.where(m, 1.0, 0.0)))
            accL, accC, tot = newL, newC, tot + _fold(loss)
        return accL, accC, tot

    @pl.when(i == 0)
    def _init():
        accL_ref[...] = jnp.zeros_like(accL_ref)
        accC_ref[...] = jnp.zeros_like(accC_ref)

    accL0 = [accL_ref[b] for b in range(_BINS - 1)]
    accC0 = [accC_ref[b] for b in range(_BINS - 1)]
    tot0 = accC_ref[_BINS - 1]
    accL, accC, tot = jax.lax.fori_loop(
        0, _NCH, chunk_body, (accL0, accC0, tot0)
    )
    for b in range(_BINS - 1):
        accL_ref[b] = accL[b]
        accC_ref[b] = accC[b]
    accC_ref[_BINS - 1] = tot

    @pl.when(i == _STEPS - 1)
    def _final():
        for b in range(_BINS - 1):
            out_ref[0, 0, b + 1] = jnp.sum(accC_ref[b])
            out_ref[0, 1, b + 1] = jnp.sum(accL_ref[b])
        out_ref[0, 0, 0] = 0.0
        out_ref[0, 1, 0] = jnp.sum(accC_ref[_BINS - 1])


def kernel(x, target):
    parts = pl.pallas_call(
        _ghm_kernel,
        grid=(_PAR, _STEPS),
        in_specs=[
            pl.BlockSpec((_BLK, _COLS), lambda o, i: (o * _STEPS + i, 0)),
            pl.BlockSpec((_BLK, _COLS), lambda o, i: (o * _STEPS + i, 0)),
        ],
        out_specs=pl.BlockSpec(
            (1, 2, _BINS), lambda o, i: (o, 0, 0), memory_space=pltpu.SMEM
        ),
        out_shape=jax.ShapeDtypeStruct((_PAR, 2, _BINS), jnp.float32),
        scratch_shapes=[
            pltpu.VMEM((_BINS, 8, 128), jnp.float32),
            pltpu.VMEM((_BINS, 8, 128), jnp.float32),
        ],
        compiler_params=pltpu.CompilerParams(
            dimension_semantics=("parallel", "arbitrary"),
        ),
    )(x, target)
    cnt = parts[:, 0, :].sum(axis=0)
    ls = parts[:, 1, :].sum(axis=0)
    m = jnp.float32(_ROWS * _COLS)
    cnt = cnt.at[0].set(m - jnp.sum(cnt))
    ls = ls.at[0].set(ls[0] - jnp.sum(ls[1:]))
    ne = jnp.sum(cnt > 0.0).astype(jnp.float32)
    gd = jnp.maximum(cnt * ne, 1e-6)
    beta = jnp.float32(_ROWS) / gd
    return jnp.sum(beta * ls) / m


# speedup vs baseline: 1.0681x; 1.0447x over previous
import jax
import jax.numpy as jnp
import numpy as np
from jax.experimental import pallas as pl
from jax.experimental.pallas import tpu as pltpu

_BINS = 10
_ROWS = 16384
_COLS = 1024
_BLK = 512
_STEPS = _ROWS // _BLK
_CH = 16
_NCH = _BLK // _CH
_SCALE = float(np.float32(_BINS - 0.0001))


def _fold(v):
    acc = v[:, 0:128]
    for k in range(1, _COLS // 128):
        acc = acc + v[:, k * 128:(k + 1) * 128]
    while acc.shape[0] > 8:
        h = acc.shape[0] // 2
        acc = acc[0:h, :] + acc[h:, :]
    return acc


def _ghm_kernel(x_ref, t_ref, out_ref, accL_ref, accC_ref):
    i = pl.program_id(0)

    def chunk_body(c, carry):
        accL, accC, tot = carry
        x = x_ref[pl.ds(c * _CH, _CH), :]
        t = t_ref[pl.ds(c * _CH, _CH), :]
        enax = jnp.exp(-jnp.abs(x))
        r = 1.0 / (1.0 + enax)
        sg = jnp.where(x >= 0.0, r, 1.0 - r)
        y = jnp.abs(sg - t) * _SCALE
        fy = jnp.floor(y)
        loss = jnp.maximum(x, 0.0) - x * t + jnp.log1p(enax)
        newL, newC = [], []
        for b in range(1, _BINS):
            m = fy == float(b)
            newL.append(accL[b - 1] + _fold(jnp.where(m, loss, 0.0)))
            newC.append(accC[b - 1] + _fold(jnp.where(m, 1.0, 0.0)))
        return newL, newC, tot + _fold(loss)

    @pl.when(i == 0)
    def _init():
        accL_ref[...] = jnp.zeros_like(accL_ref)
        accC_ref[...] = jnp.zeros_like(accC_ref)

    accL0 = [accL_ref[b] for b in range(_BINS)]
    accC0 = [accC_ref[b] for b in range(_BINS - 1)]
    tot0 = accC_ref[_BINS - 1]
    accL, accC, tot = jax.lax.fori_loop(
        0, _NCH, chunk_body, (accL0[: _BINS - 1], accC0, tot0)
    )
    for b in range(_BINS - 1):
        accL_ref[b] = accL[b]
        accC_ref[b] = accC[b]
    accC_ref[_BINS - 1] = tot

    @pl.when(i == _STEPS - 1)
    def _final():
        cs = [jnp.sum(accC_ref[b]) for b in range(_BINS - 1)]
        ls = [jnp.sum(accL_ref[b]) for b in range(_BINS - 1)]
        ltot = jnp.sum(accC_ref[_BINS - 1])
        c0 = jnp.float32(_ROWS * _COLS)
        l0 = ltot
        for c, l in zip(cs, ls):
            c0 = c0 - c
            l0 = l0 - l
        cs = [c0] + cs
        ls = [l0] + ls
        ne = c0 * 0.0
        for c in cs:
            ne = ne + jnp.where(c > 0.0, 1.0, 0.0)
        acc = c0 * 0.0
        for c, l in zip(cs, ls):
            gd = jnp.maximum(c * ne, 1e-6)
            acc = acc + (jnp.float32(_ROWS) / gd) * l
        out_ref[0, 0] = acc / jnp.float32(_ROWS * _COLS)


def kernel(x, target):
    out = pl.pallas_call(
        _ghm_kernel,
        grid=(_STEPS,),
        in_specs=[
            pl.BlockSpec((_BLK, _COLS), lambda i: (i, 0)),
            pl.BlockSpec((_BLK, _COLS), lambda i: (i, 0)),
        ],
        out_specs=pl.BlockSpec(
            (1, 1), lambda i: (0, 0), memory_space=pltpu.SMEM
        ),
        out_shape=jax.ShapeDtypeStruct((1, 1), jnp.float32),
        scratch_shapes=[
            pltpu.VMEM((_BINS, 8, 128), jnp.float32),
            pltpu.VMEM((_BINS, 8, 128), jnp.float32),
        ],
        compiler_params=pltpu.CompilerParams(
            dimension_semantics=("arbitrary",),
        ),
    )(x, target)
    return out[0, 0]


# speedup vs baseline: 1.1549x; 1.0812x over previous
import jax
import jax.numpy as jnp
import numpy as np
from jax.experimental import pallas as pl
from jax.experimental.pallas import tpu as pltpu

_BINS = 10
_ROWS = 16384
_COLS = 1024
_BLK = 512
_STEPS = _ROWS // _BLK
_CH = 16
_NCH = _BLK // _CH
_SCALE = float(np.float32(_BINS - 0.0001))


def _fold(v):
    acc = v[:, 0:128]
    for k in range(1, _COLS // 128):
        acc = acc + v[:, k * 128:(k + 1) * 128]
    while acc.shape[0] > 8:
        h = acc.shape[0] // 2
        acc = acc[0:h, :] + acc[h:, :]
    return acc


def _ghm_kernel(x_ref, t_ref, out_ref, accL_ref, accC_ref):
    i = pl.program_id(0)

    def chunk_body(c, carry):
        accL, accC, tot = carry
        x = x_ref[pl.ds(c * _CH, _CH), :]
        t = t_ref[pl.ds(c * _CH, _CH), :]
        enax = jnp.exp(-jnp.abs(x))
        one_p = 1.0 + enax
        r = 1.0 / one_p
        sg = jnp.where(x >= 0.0, r, 1.0 - r)
        y = jnp.abs(sg - t) * _SCALE
        fy = jnp.floor(y)
        loss = jnp.maximum(x, 0.0) - x * t + jnp.log(one_p)
        newL, newC = [], []
        for b in range(1, _BINS):
            m = fy == float(b)
            newL.append(accL[b - 1] + _fold(jnp.where(m, loss, 0.0)))
            newC.append(accC[b - 1] + _fold(jnp.where(m, 1.0, 0.0)))
        return newL, newC, tot + _fold(loss)

    @pl.when(i == 0)
    def _init():
        accL_ref[...] = jnp.zeros_like(accL_ref)
        accC_ref[...] = jnp.zeros_like(accC_ref)

    accL0 = [accL_ref[b] for b in range(_BINS)]
    accC0 = [accC_ref[b] for b in range(_BINS - 1)]
    tot0 = accC_ref[_BINS - 1]
    accL, accC, tot = jax.lax.fori_loop(
        0, _NCH, chunk_body, (accL0[: _BINS - 1], accC0, tot0)
    )
    for b in range(_BINS - 1):
        accL_ref[b] = accL[b]
        accC_ref[b] = accC[b]
    accC_ref[_BINS - 1] = tot

    @pl.when(i == _STEPS - 1)
    def _final():
        cs = [jnp.sum(accC_ref[b]) for b in range(_BINS - 1)]
        ls = [jnp.sum(accL_ref[b]) for b in range(_BINS - 1)]
        ltot = jnp.sum(accC_ref[_BINS - 1])
        c0 = jnp.float32(_ROWS * _COLS)
        l0 = ltot
        for c, l in zip(cs, ls):
            c0 = c0 - c
            l0 = l0 - l
        cs = [c0] + cs
        ls = [l0] + ls
        ne = c0 * 0.0
        for c in cs:
            ne = ne + jnp.where(c > 0.0, 1.0, 0.0)
        acc = c0 * 0.0
        for c, l in zip(cs, ls):
            gd = jnp.maximum(c * ne, 1e-6)
            acc = acc + (jnp.float32(_ROWS) / gd) * l
        out_ref[0, 0] = acc / jnp.float32(_ROWS * _COLS)


def kernel(x, target):
    out = pl.pallas_call(
        _ghm_kernel,
        grid=(_STEPS,),
        in_specs=[
            pl.BlockSpec((_BLK, _COLS), lambda i: (i, 0)),
            pl.BlockSpec((_BLK, _COLS), lambda i: (i, 0)),
        ],
        out_specs=pl.BlockSpec(
            (1, 1), lambda i: (0, 0), memory_space=pltpu.SMEM
        ),
        out_shape=jax.ShapeDtypeStruct((1, 1), jnp.float32),
        scratch_shapes=[
            pltpu.VMEM((_BINS, 8, 128), jnp.float32),
            pltpu.VMEM((_BINS, 8, 128), jnp.float32),
        ],
        compiler_params=pltpu.CompilerParams(
            dimension_semantics=("arbitrary",),
        ),
    )(x, target)
    return out[0, 0]


# speedup vs baseline: 1.1549x; 1.0000x over previous
import jax
import jax.numpy as jnp
import numpy as np
from jax.experimental import pallas as pl
from jax.experimental.pallas import tpu as pltpu

_BINS = 10
_ROWS = 16384
_COLS = 1024
_BLK = 512
_STEPS = _ROWS // _BLK
_CH = 16
_NCH = _BLK // _CH
_SCALE = float(np.float32(_BINS - 0.0001))


def _fold(v):
    acc = v[:, 0:128]
    for k in range(1, _COLS // 128):
        acc = acc + v[:, k * 128:(k + 1) * 128]
    while acc.shape[0] > 8:
        h = acc.shape[0] // 2
        acc = acc[0:h, :] + acc[h:, :]
    return acc


def _ghm_kernel(x_ref, t_ref, out_ref, accL_ref, accC_ref):
    i = pl.program_id(0)

    def chunk_body(c, carry):
        accL, accC, tot = carry
        x = x_ref[pl.ds(c * _CH, _CH), :]
        t = t_ref[pl.ds(c * _CH, _CH), :]
        enax = jnp.exp(-jnp.abs(x))
        one_p = 1.0 + enax
        r = pl.reciprocal(one_p, approx=True)
        sg = jnp.where(x >= 0.0, r, 1.0 - r)
        y = jnp.abs(sg - t) * _SCALE
        fy = jnp.floor(y)
        loss = jnp.maximum(x, 0.0) - x * t + jnp.log(one_p)
        newL, newC = [], []
        for b in range(1, _BINS):
            m = fy == float(b)
            newL.append(accL[b - 1] + _fold(jnp.where(m, loss, 0.0)))
            newC.append(accC[b - 1] + _fold(jnp.where(m, 1.0, 0.0)))
        return newL, newC, tot + _fold(loss)

    @pl.when(i == 0)
    def _init():
        accL_ref[...] = jnp.zeros_like(accL_ref)
        accC_ref[...] = jnp.zeros_like(accC_ref)

    accL0 = [accL_ref[b] for b in range(_BINS)]
    accC0 = [accC_ref[b] for b in range(_BINS - 1)]
    tot0 = accC_ref[_BINS - 1]
    accL, accC, tot = jax.lax.fori_loop(
        0, _NCH, chunk_body, (accL0[: _BINS - 1], accC0, tot0)
    )
    for b in range(_BINS - 1):
        accL_ref[b] = accL[b]
        accC_ref[b] = accC[b]
    accC_ref[_BINS - 1] = tot

    @pl.when(i == _STEPS - 1)
    def _final():
        cs = [jnp.sum(accC_ref[b]) for b in range(_BINS - 1)]
        ls = [jnp.sum(accL_ref[b]) for b in range(_BINS - 1)]
        ltot = jnp.sum(accC_ref[_BINS - 1])
        c0 = jnp.float32(_ROWS * _COLS)
        l0 = ltot
        for c, l in zip(cs, ls):
            c0 = c0 - c
            l0 = l0 - l
        cs = [c0] + cs
        ls = [l0] + ls
        ne = c0 * 0.0
        for c in cs:
            ne = ne + jnp.where(c > 0.0, 1.0, 0.0)
        acc = c0 * 0.0
        for c, l in zip(cs, ls):
            gd = jnp.maximum(c * ne, 1e-6)
            acc = acc + (jnp.float32(_ROWS) / gd) * l
        out_ref[0, 0] = acc / jnp.float32(_ROWS * _COLS)


def kernel(x, target):
    out = pl.pallas_call(
        _ghm_kernel,
        grid=(_STEPS,),
        in_specs=[
            pl.BlockSpec((_BLK, _COLS), lambda i: (i, 0)),
            pl.BlockSpec((_BLK, _COLS), lambda i: (i, 0)),
        ],
        out_specs=pl.BlockSpec(
            (1, 1), lambda i: (0, 0), memory_space=pltpu.SMEM
        ),
        out_shape=jax.ShapeDtypeStruct((1, 1), jnp.float32),
        scratch_shapes=[
            pltpu.VMEM((_BINS, 8, 128), jnp.float32),
            pltpu.VMEM((_BINS, 8, 128), jnp.float32),
        ],
        compiler_params=pltpu.CompilerParams(
            dimension_semantics=("arbitrary",),
        ),
    )(x, target)
    return out[0, 0]


# speedup vs baseline: 1.2447x; 1.0778x over previous
import jax
import jax.numpy as jnp
import numpy as np
from jax.experimental import pallas as pl
from jax.experimental.pallas import tpu as pltpu

_BINS = 10
_ROWS = 16384
_COLS = 1024
_BLK = 512
_STEPS = _ROWS // _BLK
_CH = 16
_NCH = _BLK // _CH
_SCALE = float(np.float32(_BINS - 0.0001))


def _fold(v):
    acc = v[:, 0:128]
    for k in range(1, _COLS // 128):
        acc = acc + v[:, k * 128:(k + 1) * 128]
    while acc.shape[0] > 8:
        h = acc.shape[0] // 2
        acc = acc[0:h, :] + acc[h:, :]
    return acc


def _digits(v):
    ds = []
    r = v
    for k in range(4, 0, -1):
        q = jnp.floor(r * (2.0 ** (-5 * k)))
        r = r - q * (2.0 ** (5 * k))
        ds.append(q)
    ds.append(r)
    return ds


def _ghm_kernel(x_ref, t_ref, out_ref, accL_ref, accC_ref):
    i = pl.program_id(0)

    def chunk_body(c, carry):
        accL, accC, tot = carry
        x = x_ref[pl.ds(c * _CH, _CH), :]
        t = t_ref[pl.ds(c * _CH, _CH), :]
        enax = jnp.exp(-jnp.abs(x))
        one_p = 1.0 + enax
        r = 1.0 / one_p
        sg = jnp.where(x >= 0.0, r, 1.0 - r)
        y = jnp.abs(sg - t) * _SCALE
        fy = jnp.floor(y)
        loss = jnp.maximum(x, 0.0) - x * t + jnp.log(one_p)
        ebits = (fy * np.float32(5 * 2 ** 23)
                 + np.float32(127 * 2 ** 23)).astype(jnp.int32)
        e = jax.lax.bitcast_convert_type(ebits, jnp.float32)
        eA = jnp.where(fy <= 4.0, e, 0.0)
        eB = (e - eA) * (2.0 ** -25)
        dA = _digits(_fold(eA))
        dB = _digits(_fold(eB))
        newC = [
            accC[0] + dA[3], accC[1] + dA[2], accC[2] + dA[1],
            accC[3] + dA[0],
            accC[4] + dB[4], accC[5] + dB[3], accC[6] + dB[2],
            accC[7] + dB[1], accC[8] + dB[0],
        ]
        newL = []
        for b in range(1, _BINS):
            m = fy == float(b)
            newL.append(accL[b - 1] + _fold(jnp.where(m, loss, 0.0)))
        return newL, newC, tot + _fold(loss)

    @pl.when(i == 0)
    def _init():
        accL_ref[...] = jnp.zeros_like(accL_ref)
        accC_ref[...] = jnp.zeros_like(accC_ref)

    accL0 = [accL_ref[b] for b in range(_BINS - 1)]
    accC0 = [accC_ref[b] for b in range(_BINS - 1)]
    tot0 = accC_ref[_BINS - 1]
    accL, accC, tot = jax.lax.fori_loop(
        0, _NCH, chunk_body, (accL0, accC0, tot0)
    )
    for b in range(_BINS - 1):
        accL_ref[b] = accL[b]
        accC_ref[b] = accC[b]
    accC_ref[_BINS - 1] = tot

    @pl.when(i == _STEPS - 1)
    def _final():
        cs = [jnp.sum(accC_ref[b]) for b in range(_BINS - 1)]
        ls = [jnp.sum(accL_ref[b]) for b in range(_BINS - 1)]
        ltot = jnp.sum(accC_ref[_BINS - 1])
        c0 = jnp.float32(_ROWS * _COLS)
        l0 = ltot
        for c, l in zip(cs, ls):
            c0 = c0 - c
            l0 = l0 - l
        cs = [c0] + cs
        ls = [l0] + ls
        ne = c0 * 0.0
        for c in cs:
            ne = ne + jnp.where(c > 0.0, 1.0, 0.0)
        acc = c0 * 0.0
        for c, l in zip(cs, ls):
            gd = jnp.maximum(c * ne, 1e-6)
            acc = acc + (jnp.float32(_ROWS) / gd) * l
        out_ref[0, 0] = acc / jnp.float32(_ROWS * _COLS)


def kernel(x, target):
    out = pl.pallas_call(
        _ghm_kernel,
        grid=(_STEPS,),
        in_specs=[
            pl.BlockSpec((_BLK, _COLS), lambda i: (i, 0)),
            pl.BlockSpec((_BLK, _COLS), lambda i: (i, 0)),
        ],
        out_specs=pl.BlockSpec(
            (1, 1), lambda i: (0, 0), memory_space=pltpu.SMEM
        ),
        out_shape=jax.ShapeDtypeStruct((1, 1), jnp.float32),
        scratch_shapes=[
            pltpu.VMEM((_BINS, 8, 128), jnp.float32),
            pltpu.VMEM((_BINS, 8, 128), jnp.float32),
        ],
        compiler_params=pltpu.CompilerParams(
            dimension_semantics=("arbitrary",),
        ),
    )(x, target)
    return out[0, 0]
